# Initial kernel scaffold; baseline (speedup 1.0000x reference)
import jax
import jax.numpy as jnp
from jax.experimental import pallas as pl


def kernel(node_repesentations, edges, edge_features, Wp0, bp0, Wp1, bp1, Wu0, bu0, Wu1, bu1):
    raise NotImplementedError("write your pallas kernel here")


# speedup vs baseline: 2.6260x; 2.6260x over previous
import functools

import jax
import jax.numpy as jnp
from jax import lax
from jax.experimental import pallas as pl
from jax.experimental.pallas import tpu as pltpu
from jax.experimental.pallas import tpu_sc as plsc

N = 10000
D = 128
DE = 16

NCORES = 2
NSUB = 16
NW = NCORES * NSUB
CH = 128

N_PAD = 10240
E_PAD = 323584
RPW = E_PAD // NW
NCHUNK = RPW // CH
RPS = N_PAD // NSUB
BE = 2048
BN = 2048


def _gelu(x):
    return jax.nn.gelu(x)


def _proj_body(x_ref, w_ref, o_ref):
    o_ref[...] = jnp.dot(x_ref[...], w_ref[...], preferred_element_type=jnp.float32)


def _project(nodes_pad, w):
    return pl.pallas_call(
        _proj_body,
        out_shape=jax.ShapeDtypeStruct((N_PAD, D), jnp.float32),
    )(nodes_pad, w)


def _sc_gather(table, idx):
    mesh = plsc.VectorSubcoreMesh(core_axis_name="c", subcore_axis_name="s")

    @functools.partial(
        pl.kernel,
        out_type=jax.ShapeDtypeStruct((E_PAD, D), jnp.float32),
        mesh=mesh,
        compiler_params=pltpu.CompilerParams(needs_layout_passes=False),
        scratch_types=[
            pltpu.VMEM((CH,), jnp.int32),
            pltpu.VMEM((CH, D), jnp.float32),
            pltpu.SemaphoreType.DMA,
        ],
    )
    def k(table_hbm, idx_hbm, out_hbm, idx_v, rows_v, sem):
        wid = lax.axis_index("s") * NCORES + lax.axis_index("c")
        base = wid * RPW

        def body(i, carry):
            off = base + i * CH
            pltpu.sync_copy(idx_hbm.at[pl.ds(off, CH)], idx_v)
            pltpu.async_copy(table_hbm.at[idx_v], rows_v, sem).wait()
            pltpu.sync_copy(rows_v, out_hbm.at[pl.ds(off, CH), :])
            return carry

        lax.fori_loop(0, NCHUNK, body, 0)

    return k(table, idx)


def _edge_ffn_body(g_ref, ef_ref, w0e_ref, b0_ref, w1_ref, b1_ref, o_ref):
    pre = (
        g_ref[...]
        + jnp.dot(ef_ref[...], w0e_ref[...], preferred_element_type=jnp.float32)
        + b0_ref[...]
    )
    h = _gelu(pre)
    o_ref[...] = _gelu(
        jnp.dot(h, w1_ref[...], preferred_element_type=jnp.float32) + b1_ref[...]
    )


def _edge_ffn(g, ef_pad, w0e, b0, w1, b1):
    nblk = E_PAD // BE
    return pl.pallas_call(
        _edge_ffn_body,
        grid=(nblk,),
        in_specs=[
            pl.BlockSpec((BE, D), lambda i: (i, 0)),
            pl.BlockSpec((BE, DE), lambda i: (i, 0)),
            pl.BlockSpec((DE, D), lambda i: (0, 0)),
            pl.BlockSpec((1, D), lambda i: (0, 0)),
            pl.BlockSpec((D, D), lambda i: (0, 0)),
            pl.BlockSpec((1, D), lambda i: (0, 0)),
        ],
        out_specs=pl.BlockSpec((BE, D), lambda i: (i, 0)),
        out_shape=jax.ShapeDtypeStruct((E_PAD, D), jnp.float32),
    )(g, ef_pad, w0e, b0, w1, b1)


def _sc_scatter(msgs, dst):
    mesh = plsc.VectorSubcoreMesh(core_axis_name="c", subcore_axis_name="s")

    @functools.partial(
        pl.kernel,
        out_type=(
            jax.ShapeDtypeStruct((NCORES, N_PAD, D), jnp.float32),
            jax.ShapeDtypeStruct((NW, N_PAD), jnp.float32),
        ),
        mesh=mesh,
        compiler_params=pltpu.CompilerParams(needs_layout_passes=False),
        scratch_types=[
            pltpu.VMEM((CH,), jnp.int32),
            pltpu.VMEM((CH, D), jnp.float32),
            pltpu.VMEM((N_PAD,), jnp.float32),
            pltpu.VMEM_SHARED((N_PAD, D), jnp.float32),
            pltpu.SemaphoreType.DMA,
        ],
    )
    def k(msgs_hbm, dst_hbm, sums_hbm, counts_hbm, idx_v, buf_v, cnt_v, acc_sh, sem):
        cid = lax.axis_index("c")
        sid = lax.axis_index("s")
        wid = sid * NCORES + cid
        z16 = jnp.zeros((16,), jnp.float32)

        def zbuf(i, carry):
            r = i // (D // 16)
            c = (i % (D // 16)) * 16
            buf_v[r, pl.ds(c, 16)] = z16
            return carry

        lax.fori_loop(0, CH * (D // 16), zbuf, 0)

        def zcnt(i, carry):
            cnt_v[pl.ds(i * 16, 16)] = z16
            return carry

        lax.fori_loop(0, N_PAD // 16, zcnt, 0)

        for j in range(RPS // CH):
            pltpu.sync_copy(buf_v, acc_sh.at[pl.ds(sid * RPS + j * CH, CH), :])
        plsc.subcore_barrier()

        ones16 = jnp.ones((16,), jnp.float32)
        base = wid * RPW

        def body(i, carry):
            off = base + i * CH
            pltpu.sync_copy(dst_hbm.at[pl.ds(off, CH)], idx_v)
            pltpu.sync_copy(msgs_hbm.at[pl.ds(off, CH), :], buf_v)
            pltpu.sync_copy(buf_v, acc_sh.at[idx_v], add=True)
            def cbody(j, c2):
                iv = idx_v[pl.ds(j * 16, 16)]
                plsc.addupdate_scatter(cnt_v, [iv], ones16)
                return c2

            lax.fori_loop(0, CH // 16, cbody, 0)
            return carry

        lax.fori_loop(0, NCHUNK, body, 0)
        plsc.subcore_barrier()

        for j in range(RPS // CH):
            r0 = sid * RPS + j * CH
            pltpu.sync_copy(acc_sh.at[pl.ds(r0, CH), :], buf_v)
            pltpu.sync_copy(buf_v, sums_hbm.at[cid, pl.ds(r0, CH), :])
        pltpu.sync_copy(cnt_v, counts_hbm.at[wid])

    return k(msgs, dst)


def _node_ffn_body(x_ref, s_ref, c_ref, w0a_ref, w0b_ref, b0_ref, w1_ref, b1_ref, o_ref):
    s = s_ref[0] + s_ref[1]
    c = jnp.sum(c_ref[...], axis=0)
    agg = s / jnp.maximum(c, 1.0)[:, None]
    h = _gelu(
        jnp.dot(x_ref[...], w0a_ref[...], preferred_element_type=jnp.float32)
        + jnp.dot(agg, w0b_ref[...], preferred_element_type=jnp.float32)
        + b0_ref[...]
    )
    o_ref[...] = _gelu(
        jnp.dot(h, w1_ref[...], preferred_element_type=jnp.float32) + b1_ref[...]
    )


def _node_ffn(nodes_pad, sums, counts, w0a, w0b, b0, w1, b1):
    nblk = N_PAD // BN
    return pl.pallas_call(
        _node_ffn_body,
        grid=(nblk,),
        in_specs=[
            pl.BlockSpec((BN, D), lambda i: (i, 0)),
            pl.BlockSpec((NCORES, BN, D), lambda i: (0, i, 0)),
            pl.BlockSpec((NW, BN), lambda i: (0, i)),
            pl.BlockSpec((D, D), lambda i: (0, 0)),
            pl.BlockSpec((D, D), lambda i: (0, 0)),
            pl.BlockSpec((1, D), lambda i: (0, 0)),
            pl.BlockSpec((D, D), lambda i: (0, 0)),
            pl.BlockSpec((1, D), lambda i: (0, 0)),
        ],
        out_specs=pl.BlockSpec((BN, D), lambda i: (i, 0)),
        out_shape=jax.ShapeDtypeStruct((N_PAD, D), jnp.float32),
    )(nodes_pad, sums, counts, w0a, w0b, b0, w1, b1)


def kernel(node_repesentations, edges, edge_features, Wp0, bp0, Wp1, bp1, Wu0, bu0, Wu1, bu1):
    nodes_pad = jnp.pad(node_repesentations, ((0, N_PAD - N), (0, 0)))
    src = jnp.pad(edges[0].astype(jnp.int32), (0, E_PAD - edges.shape[1]))
    dst = jnp.pad(
        edges[1].astype(jnp.int32),
        (0, E_PAD - edges.shape[1]),
        constant_values=N_PAD - 1,
    )
    ef_pad = jnp.pad(edge_features, ((0, E_PAD - edge_features.shape[0]), (0, 0)))

    p = _project(nodes_pad, Wp0[DE:])
    g = _sc_gather(p, src)
    msgs = _edge_ffn(g, ef_pad, Wp0[:DE], bp0.reshape(1, D), Wp1, bp1.reshape(1, D))
    sums, counts = _sc_scatter(msgs, dst)
    out = _node_ffn(
        nodes_pad, sums, counts,
        Wu0[:D], Wu0[D:], bu0.reshape(1, D), Wu1, bu1.reshape(1, D),
    )
    return out[:N]
